# baseline (speedup 1.0000x reference)
import jax
import jax.numpy as jnp
from jax.experimental import pallas as pl
from jax.experimental.pallas import tpu as pltpu

_K = 10
_B = 128
_V = 32768
_RG = 16
_LANES = 128
_CHUNK = 32768
_NC = _V // _CHUNK
_NSL = _CHUNK // _LANES
_NG = _B // _RG
_NEG = -3.0e38


def _insert_chunk(llm_ref, slm_ref, til_ref, tis_ref, first):
    if first:
        init_l = tuple(jnp.full((_RG, _LANES), _NEG, dtype=jnp.float32)
                       for _ in range(_K))
        init_s = init_l
    else:
        init_l = tuple(til_ref[:, k * _LANES:(k + 1) * _LANES]
                       for k in range(_K))
        init_s = tuple(tis_ref[:, k * _LANES:(k + 1) * _LANES]
                       for k in range(_K))

    def body(i, carry):
        tl, ts = list(carry[0]), list(carry[1])
        vl = llm_ref[:, pl.ds(i * _LANES, _LANES)]
        vs = slm_ref[:, pl.ds(i * _LANES, _LANES)]
        for k in range(_K):
            hl = jnp.maximum(tl[k], vl)
            vl = jnp.minimum(tl[k], vl)
            tl[k] = hl
            hs = jnp.maximum(ts[k], vs)
            vs = jnp.minimum(ts[k], vs)
            ts[k] = hs
        return tuple(tl), tuple(ts)

    tl, ts = jax.lax.fori_loop(0, _NSL, body, (init_l, init_s), unroll=8)
    for k in range(_K):
        til_ref[:, k * _LANES:(k + 1) * _LANES] = tl[k]
        tis_ref[:, k * _LANES:(k + 1) * _LANES] = ts[k]


def _merge_topk(tiles_ref):
    cand = tiles_ref[...]
    idx = jax.lax.broadcasted_iota(jnp.int32, cand.shape, 1)
    outs = []
    for _ in range(_K):
        m = jnp.max(cand, axis=1, keepdims=True)
        eq = cand == m
        pos = jnp.min(jnp.where(eq, idx, _K * _LANES), axis=1, keepdims=True)
        cand = jnp.where(idx == pos, _NEG, cand)
        outs.append(m)
    return jnp.concatenate(outs, axis=1)


def _topk_body(llm_ref, slm_ref, otl_ref, ots_ref, til_ref, tis_ref):
    c = pl.program_id(1)

    @pl.when(c == 0)
    def _():
        _insert_chunk(llm_ref, slm_ref, til_ref, tis_ref, True)

    @pl.when(c != 0)
    def _():
        _insert_chunk(llm_ref, slm_ref, til_ref, tis_ref, False)

    @pl.when(c == _NC - 1)
    def _():
        otl_ref[...] = _merge_topk(til_ref)
        ots_ref[...] = _merge_topk(tis_ref)


def _r16(x):
    b = jax.lax.bitcast_convert_type(x, jnp.int32)
    b = (b + 0xFFF + ((b >> 13) & 1)) & ~0x1FFF
    return jax.lax.bitcast_convert_type(b, jnp.float32)


def _mlp_body(tl_ref, ts_ref, w1t_ref, b1_ref, w2t_ref, b2_ref,
              w3t_ref, b3_ref, out_ref):
    c = jnp.concatenate([tl_ref[...], ts_ref[...]], axis=1)
    z1 = _r16(_r16(jnp.dot(c, w1t_ref[...],
                           preferred_element_type=jnp.float32)) + b1_ref[...])
    h1 = jnp.maximum(z1, 0.0)
    z2 = _r16(_r16(jnp.dot(h1, w2t_ref[...],
                           preferred_element_type=jnp.float32)) + b2_ref[...])
    h2 = jnp.maximum(z2, 0.0)
    z3 = _r16(_r16(jnp.dot(h2, w3t_ref[...],
                           preferred_element_type=jnp.float32)) + b3_ref[...])
    raw = _r16(jax.nn.sigmoid(z3))
    s = _r16(jnp.sum(raw, axis=1, keepdims=True))
    out_ref[...] = raw / s


def kernel(llm_logits, slm_logits, W1, b1, W2, b2, W3, b3):
    llm32 = llm_logits.astype(jnp.float32)
    slm32 = slm_logits.astype(jnp.float32)

    tl, ts = pl.pallas_call(
        _topk_body,
        grid=(_NG, _NC),
        in_specs=[
            pl.BlockSpec((_RG, _CHUNK), lambda g, c: (g, c)),
            pl.BlockSpec((_RG, _CHUNK), lambda g, c: (g, c)),
        ],
        out_specs=[
            pl.BlockSpec((_RG, _K), lambda g, c: (g, 0)),
            pl.BlockSpec((_RG, _K), lambda g, c: (g, 0)),
        ],
        out_shape=[
            jax.ShapeDtypeStruct((_B, _K), jnp.float32),
            jax.ShapeDtypeStruct((_B, _K), jnp.float32),
        ],
        scratch_shapes=[
            pltpu.VMEM((_RG, _K * _LANES), jnp.float32),
            pltpu.VMEM((_RG, _K * _LANES), jnp.float32),
        ],
        compiler_params=pltpu.CompilerParams(
            dimension_semantics=("parallel", "arbitrary")),
    )(llm32, slm32)

    w1t = W1.T.astype(jnp.float32)
    w2t = W2.T.astype(jnp.float32)
    w3t = W3.T.astype(jnp.float32)
    b1r = b1.reshape(1, -1).astype(jnp.float32)
    b2r = b2.reshape(1, -1).astype(jnp.float32)
    b3r = b3.reshape(1, -1).astype(jnp.float32)

    full = lambda shape: pl.BlockSpec(shape, lambda: (0,) * len(shape))
    out = pl.pallas_call(
        _mlp_body,
        in_specs=[full((_B, _K)), full((_B, _K)),
                  full(w1t.shape), full(b1r.shape),
                  full(w2t.shape), full(b2r.shape),
                  full(w3t.shape), full(b3r.shape)],
        out_specs=full((_B, 2)),
        out_shape=jax.ShapeDtypeStruct((_B, 2), jnp.float32),
    )(tl, ts, w1t, b1r, w2t, b2r, w3t, b3r)
    return out.astype(jnp.float16)


# speedup vs baseline: 1.0111x; 1.0111x over previous
import jax
import jax.numpy as jnp
from jax.experimental import pallas as pl
from jax.experimental.pallas import tpu as pltpu

_K = 10
_B = 128
_V = 32768
_RG = 16
_LANES = 128
_CHUNK = 32768
_NC = _V // _CHUNK
_NSL = _CHUNK // _LANES
_NG = _B // _RG
_NEG = -3.0e38


def _insert_chunk(llm_ref, slm_ref, til_ref, tis_ref, first):
    if first:
        init_l = tuple(jnp.full((_RG, _LANES), _NEG, dtype=jnp.float32)
                       for _ in range(_K))
        init_s = init_l
    else:
        init_l = tuple(til_ref[:, k * _LANES:(k + 1) * _LANES]
                       for k in range(_K))
        init_s = tuple(tis_ref[:, k * _LANES:(k + 1) * _LANES]
                       for k in range(_K))

    def body(i, carry):
        tl, ts = list(carry[0]), list(carry[1])
        vl = llm_ref[:, pl.ds(i * _LANES, _LANES)]
        vs = slm_ref[:, pl.ds(i * _LANES, _LANES)]
        for k in range(_K):
            hl = jnp.maximum(tl[k], vl)
            vl = jnp.minimum(tl[k], vl)
            tl[k] = hl
            hs = jnp.maximum(ts[k], vs)
            vs = jnp.minimum(ts[k], vs)
            ts[k] = hs
        return tuple(tl), tuple(ts)

    tl, ts = jax.lax.fori_loop(0, _NSL, body, (init_l, init_s), unroll=16)
    for k in range(_K):
        til_ref[:, k * _LANES:(k + 1) * _LANES] = tl[k]
        tis_ref[:, k * _LANES:(k + 1) * _LANES] = ts[k]


def _merge_topk(tiles_ref):
    cand = tiles_ref[...]
    idx = jax.lax.broadcasted_iota(jnp.int32, cand.shape, 1)
    outs = []
    for _ in range(_K):
        m = jnp.max(cand, axis=1, keepdims=True)
        eq = cand == m
        pos = jnp.min(jnp.where(eq, idx, _K * _LANES), axis=1, keepdims=True)
        cand = jnp.where(idx == pos, _NEG, cand)
        outs.append(m)
    return jnp.concatenate(outs, axis=1)


def _topk_body(llm_ref, slm_ref, otl_ref, ots_ref, til_ref, tis_ref):
    c = pl.program_id(1)

    @pl.when(c == 0)
    def _():
        _insert_chunk(llm_ref, slm_ref, til_ref, tis_ref, True)

    @pl.when(c != 0)
    def _():
        _insert_chunk(llm_ref, slm_ref, til_ref, tis_ref, False)

    @pl.when(c == _NC - 1)
    def _():
        otl_ref[...] = _merge_topk(til_ref)
        ots_ref[...] = _merge_topk(tis_ref)


def _r16(x):
    b = jax.lax.bitcast_convert_type(x, jnp.int32)
    b = (b + 0xFFF + ((b >> 13) & 1)) & ~0x1FFF
    return jax.lax.bitcast_convert_type(b, jnp.float32)


def _mlp_body(tl_ref, ts_ref, w1t_ref, b1_ref, w2t_ref, b2_ref,
              w3t_ref, b3_ref, out_ref):
    c = jnp.concatenate([tl_ref[...], ts_ref[...]], axis=1)
    z1 = _r16(_r16(jnp.dot(c, w1t_ref[...],
                           preferred_element_type=jnp.float32)) + b1_ref[...])
    h1 = jnp.maximum(z1, 0.0)
    z2 = _r16(_r16(jnp.dot(h1, w2t_ref[...],
                           preferred_element_type=jnp.float32)) + b2_ref[...])
    h2 = jnp.maximum(z2, 0.0)
    z3 = _r16(_r16(jnp.dot(h2, w3t_ref[...],
                           preferred_element_type=jnp.float32)) + b3_ref[...])
    raw = _r16(jax.nn.sigmoid(z3))
    s = _r16(jnp.sum(raw, axis=1, keepdims=True))
    out_ref[...] = raw / s


def _topk_stage(llm32, slm32):
    return pl.pallas_call(
        _topk_body,
        grid=(_NG, _NC),
        in_specs=[
            pl.BlockSpec((_RG, _CHUNK), lambda g, c: (g, c)),
            pl.BlockSpec((_RG, _CHUNK), lambda g, c: (g, c)),
        ],
        out_specs=[
            pl.BlockSpec((_RG, _K), lambda g, c: (g, 0)),
            pl.BlockSpec((_RG, _K), lambda g, c: (g, 0)),
        ],
        out_shape=[
            jax.ShapeDtypeStruct((_B, _K), jnp.float32),
            jax.ShapeDtypeStruct((_B, _K), jnp.float32),
        ],
        scratch_shapes=[
            pltpu.VMEM((_RG, _K * _LANES), jnp.float32),
            pltpu.VMEM((_RG, _K * _LANES), jnp.float32),
        ],
        compiler_params=pltpu.CompilerParams(
            dimension_semantics=("parallel", "arbitrary")),
    )(llm32, slm32)


def kernel(llm_logits, slm_logits, W1, b1, W2, b2, W3, b3):
    llm32 = llm_logits.astype(jnp.float32)
    slm32 = slm_logits.astype(jnp.float32)
    tl, ts = _topk_stage(llm32, slm32)

    w1t = W1.T.astype(jnp.float32)
    w2t = W2.T.astype(jnp.float32)
    w3t = W3.T.astype(jnp.float32)
    b1r = b1.reshape(1, -1).astype(jnp.float32)
    b2r = b2.reshape(1, -1).astype(jnp.float32)
    b3r = b3.reshape(1, -1).astype(jnp.float32)

    full = lambda shape: pl.BlockSpec(shape, lambda: (0,) * len(shape))
    out = pl.pallas_call(
        _mlp_body,
        in_specs=[full((_B, _K)), full((_B, _K)),
                  full(w1t.shape), full(b1r.shape),
                  full(w2t.shape), full(b2r.shape),
                  full(w3t.shape), full(b3r.shape)],
        out_specs=full((_B, 2)),
        out_shape=jax.ShapeDtypeStruct((_B, 2), jnp.float32),
    )(tl, ts, w1t, b1r, w2t, b2r, w3t, b3r)
    return out.astype(jnp.float16)


# speedup vs baseline: 1.0281x; 1.0168x over previous
import jax
import jax.numpy as jnp
from jax.experimental import pallas as pl
from jax.experimental.pallas import tpu as pltpu

_K = 10
_B = 128
_V = 32768
_RG = 16
_LANES = 128
_CHUNK = 32768
_NC = _V // _CHUNK
_NSL = _CHUNK // _LANES
_NG = _B // _RG
_NEG = -3.0e38


def _insert_chunk(llm_ref, slm_ref, til_ref, tis_ref, first):
    if first:
        init_l = tuple(jnp.full((_RG, _LANES), _NEG, dtype=jnp.float32)
                       for _ in range(_K))
        init_s = init_l
    else:
        init_l = tuple(til_ref[:, k * _LANES:(k + 1) * _LANES]
                       for k in range(_K))
        init_s = tuple(tis_ref[:, k * _LANES:(k + 1) * _LANES]
                       for k in range(_K))

    def body(i, carry):
        tl, ts = list(carry[0]), list(carry[1])
        vl = llm_ref[:, pl.ds(i * _LANES, _LANES)]
        vs = slm_ref[:, pl.ds(i * _LANES, _LANES)]
        for k in range(_K):
            hl = jnp.maximum(tl[k], vl)
            vl = jnp.minimum(tl[k], vl)
            tl[k] = hl
            hs = jnp.maximum(ts[k], vs)
            vs = jnp.minimum(ts[k], vs)
            ts[k] = hs
        return tuple(tl), tuple(ts)

    tl, ts = jax.lax.fori_loop(0, _NSL, body, (init_l, init_s), unroll=32)
    for k in range(_K):
        til_ref[:, k * _LANES:(k + 1) * _LANES] = tl[k]
        tis_ref[:, k * _LANES:(k + 1) * _LANES] = ts[k]


def _merge_topk(tiles_ref):
    cand = tiles_ref[...]
    idx = jax.lax.broadcasted_iota(jnp.int32, cand.shape, 1)
    outs = []
    for _ in range(_K):
        m = jnp.max(cand, axis=1, keepdims=True)
        eq = cand == m
        pos = jnp.min(jnp.where(eq, idx, _K * _LANES), axis=1, keepdims=True)
        cand = jnp.where(idx == pos, _NEG, cand)
        outs.append(m)
    return jnp.concatenate(outs, axis=1)


def _topk_body(llm_ref, slm_ref, otl_ref, ots_ref, til_ref, tis_ref):
    c = pl.program_id(1)

    @pl.when(c == 0)
    def _():
        _insert_chunk(llm_ref, slm_ref, til_ref, tis_ref, True)

    @pl.when(c != 0)
    def _():
        _insert_chunk(llm_ref, slm_ref, til_ref, tis_ref, False)

    @pl.when(c == _NC - 1)
    def _():
        otl_ref[...] = _merge_topk(til_ref)
        ots_ref[...] = _merge_topk(tis_ref)


def _r16(x):
    b = jax.lax.bitcast_convert_type(x, jnp.int32)
    b = (b + 0xFFF + ((b >> 13) & 1)) & ~0x1FFF
    return jax.lax.bitcast_convert_type(b, jnp.float32)


def _mlp_body(tl_ref, ts_ref, w1t_ref, b1_ref, w2t_ref, b2_ref,
              w3t_ref, b3_ref, out_ref):
    c = jnp.concatenate([tl_ref[...], ts_ref[...]], axis=1)
    z1 = _r16(_r16(jnp.dot(c, w1t_ref[...],
                           preferred_element_type=jnp.float32)) + b1_ref[...])
    h1 = jnp.maximum(z1, 0.0)
    z2 = _r16(_r16(jnp.dot(h1, w2t_ref[...],
                           preferred_element_type=jnp.float32)) + b2_ref[...])
    h2 = jnp.maximum(z2, 0.0)
    z3 = _r16(_r16(jnp.dot(h2, w3t_ref[...],
                           preferred_element_type=jnp.float32)) + b3_ref[...])
    raw = _r16(jax.nn.sigmoid(z3))
    s = _r16(jnp.sum(raw, axis=1, keepdims=True))
    out_ref[...] = raw / s


def _topk_stage(llm32, slm32):
    return pl.pallas_call(
        _topk_body,
        grid=(_NG, _NC),
        in_specs=[
            pl.BlockSpec((_RG, _CHUNK), lambda g, c: (g, c)),
            pl.BlockSpec((_RG, _CHUNK), lambda g, c: (g, c)),
        ],
        out_specs=[
            pl.BlockSpec((_RG, _K), lambda g, c: (g, 0)),
            pl.BlockSpec((_RG, _K), lambda g, c: (g, 0)),
        ],
        out_shape=[
            jax.ShapeDtypeStruct((_B, _K), jnp.float32),
            jax.ShapeDtypeStruct((_B, _K), jnp.float32),
        ],
        scratch_shapes=[
            pltpu.VMEM((_RG, _K * _LANES), jnp.float32),
            pltpu.VMEM((_RG, _K * _LANES), jnp.float32),
        ],
        compiler_params=pltpu.CompilerParams(
            dimension_semantics=("parallel", "arbitrary")),
    )(llm32, slm32)


def kernel(llm_logits, slm_logits, W1, b1, W2, b2, W3, b3):
    llm32 = llm_logits.astype(jnp.float32)
    slm32 = slm_logits.astype(jnp.float32)
    tl, ts = _topk_stage(llm32, slm32)

    w1t = W1.T.astype(jnp.float32)
    w2t = W2.T.astype(jnp.float32)
    w3t = W3.T.astype(jnp.float32)
    b1r = b1.reshape(1, -1).astype(jnp.float32)
    b2r = b2.reshape(1, -1).astype(jnp.float32)
    b3r = b3.reshape(1, -1).astype(jnp.float32)

    full = lambda shape: pl.BlockSpec(shape, lambda: (0,) * len(shape))
    out = pl.pallas_call(
        _mlp_body,
        in_specs=[full((_B, _K)), full((_B, _K)),
                  full(w1t.shape), full(b1r.shape),
                  full(w2t.shape), full(b2r.shape),
                  full(w3t.shape), full(b3r.shape)],
        out_specs=full((_B, 2)),
        out_shape=jax.ShapeDtypeStruct((_B, 2), jnp.float32),
    )(tl, ts, w1t, b1r, w2t, b2r, w3t, b3r)
    return out.astype(jnp.float16)


# speedup vs baseline: 1.0352x; 1.0069x over previous
import jax
import jax.numpy as jnp
from jax.experimental import pallas as pl
from jax.experimental.pallas import tpu as pltpu

_K = 10
_B = 128
_V = 32768
_RG = 16
_LANES = 128
_CHUNK = 32768
_NC = _V // _CHUNK
_NSL = _CHUNK // _LANES
_NG = _B // _RG
_NEG = -3.0e38


def _insert_chunk(llm_ref, slm_ref, til_ref, tis_ref, first):
    if first:
        init_l = tuple(jnp.full((_RG, _LANES), _NEG, dtype=jnp.float32)
                       for _ in range(_K))
        init_s = init_l
    else:
        init_l = tuple(til_ref[:, k * _LANES:(k + 1) * _LANES]
                       for k in range(_K))
        init_s = tuple(tis_ref[:, k * _LANES:(k + 1) * _LANES]
                       for k in range(_K))

    def body(i, carry):
        tl, ts = list(carry[0]), list(carry[1])
        vl = llm_ref[:, pl.ds(i * _LANES, _LANES)]
        vs = slm_ref[:, pl.ds(i * _LANES, _LANES)]
        for k in range(_K):
            hl = jnp.maximum(tl[k], vl)
            vl = jnp.minimum(tl[k], vl)
            tl[k] = hl
            hs = jnp.maximum(ts[k], vs)
            vs = jnp.minimum(ts[k], vs)
            ts[k] = hs
        return tuple(tl), tuple(ts)

    tl, ts = jax.lax.fori_loop(0, _NSL, body, (init_l, init_s), unroll=64)
    for k in range(_K):
        til_ref[:, k * _LANES:(k + 1) * _LANES] = tl[k]
        tis_ref[:, k * _LANES:(k + 1) * _LANES] = ts[k]


def _merge_topk(tiles_ref):
    cand = tiles_ref[...]
    idx = jax.lax.broadcasted_iota(jnp.int32, cand.shape, 1)
    outs = []
    for _ in range(_K):
        m = jnp.max(cand, axis=1, keepdims=True)
        eq = cand == m
        pos = jnp.min(jnp.where(eq, idx, _K * _LANES), axis=1, keepdims=True)
        cand = jnp.where(idx == pos, _NEG, cand)
        outs.append(m)
    return jnp.concatenate(outs, axis=1)


def _topk_body(llm_ref, slm_ref, otl_ref, ots_ref, til_ref, tis_ref):
    c = pl.program_id(1)

    @pl.when(c == 0)
    def _():
        _insert_chunk(llm_ref, slm_ref, til_ref, tis_ref, True)

    @pl.when(c != 0)
    def _():
        _insert_chunk(llm_ref, slm_ref, til_ref, tis_ref, False)

    @pl.when(c == _NC - 1)
    def _():
        otl_ref[...] = _merge_topk(til_ref)
        ots_ref[...] = _merge_topk(tis_ref)


def _r16(x):
    b = jax.lax.bitcast_convert_type(x, jnp.int32)
    b = (b + 0xFFF + ((b >> 13) & 1)) & ~0x1FFF
    return jax.lax.bitcast_convert_type(b, jnp.float32)


def _mlp_body(tl_ref, ts_ref, w1t_ref, b1_ref, w2t_ref, b2_ref,
              w3t_ref, b3_ref, out_ref):
    c = jnp.concatenate([tl_ref[...], ts_ref[...]], axis=1)
    z1 = _r16(_r16(jnp.dot(c, w1t_ref[...],
                           preferred_element_type=jnp.float32)) + b1_ref[...])
    h1 = jnp.maximum(z1, 0.0)
    z2 = _r16(_r16(jnp.dot(h1, w2t_ref[...],
                           preferred_element_type=jnp.float32)) + b2_ref[...])
    h2 = jnp.maximum(z2, 0.0)
    z3 = _r16(_r16(jnp.dot(h2, w3t_ref[...],
                           preferred_element_type=jnp.float32)) + b3_ref[...])
    raw = _r16(jax.nn.sigmoid(z3))
    s = _r16(jnp.sum(raw, axis=1, keepdims=True))
    out_ref[...] = raw / s


def _topk_stage(llm32, slm32):
    return pl.pallas_call(
        _topk_body,
        grid=(_NG, _NC),
        in_specs=[
            pl.BlockSpec((_RG, _CHUNK), lambda g, c: (g, c)),
            pl.BlockSpec((_RG, _CHUNK), lambda g, c: (g, c)),
        ],
        out_specs=[
            pl.BlockSpec((_RG, _K), lambda g, c: (g, 0)),
            pl.BlockSpec((_RG, _K), lambda g, c: (g, 0)),
        ],
        out_shape=[
            jax.ShapeDtypeStruct((_B, _K), jnp.float32),
            jax.ShapeDtypeStruct((_B, _K), jnp.float32),
        ],
        scratch_shapes=[
            pltpu.VMEM((_RG, _K * _LANES), jnp.float32),
            pltpu.VMEM((_RG, _K * _LANES), jnp.float32),
        ],
        compiler_params=pltpu.CompilerParams(
            dimension_semantics=("parallel", "arbitrary")),
    )(llm32, slm32)


def kernel(llm_logits, slm_logits, W1, b1, W2, b2, W3, b3):
    llm32 = llm_logits.astype(jnp.float32)
    slm32 = slm_logits.astype(jnp.float32)
    tl, ts = _topk_stage(llm32, slm32)

    w1t = W1.T.astype(jnp.float32)
    w2t = W2.T.astype(jnp.float32)
    w3t = W3.T.astype(jnp.float32)
    b1r = b1.reshape(1, -1).astype(jnp.float32)
    b2r = b2.reshape(1, -1).astype(jnp.float32)
    b3r = b3.reshape(1, -1).astype(jnp.float32)

    full = lambda shape: pl.BlockSpec(shape, lambda: (0,) * len(shape))
    out = pl.pallas_call(
        _mlp_body,
        in_specs=[full((_B, _K)), full((_B, _K)),
                  full(w1t.shape), full(b1r.shape),
                  full(w2t.shape), full(b2r.shape),
                  full(w3t.shape), full(b3r.shape)],
        out_specs=full((_B, 2)),
        out_shape=jax.ShapeDtypeStruct((_B, 2), jnp.float32),
    )(tl, ts, w1t, b1r, w2t, b2r, w3t, b3r)
    return out.astype(jnp.float16)


# speedup vs baseline: 1.0376x; 1.0024x over previous
import jax
import jax.numpy as jnp
from jax.experimental import pallas as pl
from jax.experimental.pallas import tpu as pltpu

_K = 10
_B = 128
_V = 32768
_RG = 16
_LANES = 128
_CHUNK = 32768
_NC = _V // _CHUNK
_NSL = _CHUNK // _LANES
_NG = _B // _RG
_NEG = -3.0e38


def _insert_chunk(llm_ref, slm_ref, til_ref, tis_ref, first):
    if first:
        init_l = tuple(jnp.full((_RG, _LANES), _NEG, dtype=jnp.float32)
                       for _ in range(_K))
        init_s = init_l
    else:
        init_l = tuple(til_ref[:, k * _LANES:(k + 1) * _LANES]
                       for k in range(_K))
        init_s = tuple(tis_ref[:, k * _LANES:(k + 1) * _LANES]
                       for k in range(_K))

    def body(i, carry):
        tl, ts = list(carry[0]), list(carry[1])
        vl = llm_ref[:, pl.ds(i * _LANES, _LANES)]
        vs = slm_ref[:, pl.ds(i * _LANES, _LANES)]
        for k in range(_K):
            hl = jnp.maximum(tl[k], vl)
            vl = jnp.minimum(tl[k], vl)
            tl[k] = hl
            hs = jnp.maximum(ts[k], vs)
            vs = jnp.minimum(ts[k], vs)
            ts[k] = hs
        return tuple(tl), tuple(ts)

    tl, ts = jax.lax.fori_loop(0, _NSL, body, (init_l, init_s), unroll=128)
    for k in range(_K):
        til_ref[:, k * _LANES:(k + 1) * _LANES] = tl[k]
        tis_ref[:, k * _LANES:(k + 1) * _LANES] = ts[k]


def _merge_topk(tiles_ref):
    cand = tiles_ref[...]
    idx = jax.lax.broadcasted_iota(jnp.int32, cand.shape, 1)
    outs = []
    for _ in range(_K):
        m = jnp.max(cand, axis=1, keepdims=True)
        eq = cand == m
        pos = jnp.min(jnp.where(eq, idx, _K * _LANES), axis=1, keepdims=True)
        cand = jnp.where(idx == pos, _NEG, cand)
        outs.append(m)
    return jnp.concatenate(outs, axis=1)


def _topk_body(llm_ref, slm_ref, otl_ref, ots_ref, til_ref, tis_ref):
    c = pl.program_id(1)

    @pl.when(c == 0)
    def _():
        _insert_chunk(llm_ref, slm_ref, til_ref, tis_ref, True)

    @pl.when(c != 0)
    def _():
        _insert_chunk(llm_ref, slm_ref, til_ref, tis_ref, False)

    @pl.when(c == _NC - 1)
    def _():
        otl_ref[...] = _merge_topk(til_ref)
        ots_ref[...] = _merge_topk(tis_ref)


def _r16(x):
    b = jax.lax.bitcast_convert_type(x, jnp.int32)
    b = (b + 0xFFF + ((b >> 13) & 1)) & ~0x1FFF
    return jax.lax.bitcast_convert_type(b, jnp.float32)


def _mlp_body(tl_ref, ts_ref, w1t_ref, b1_ref, w2t_ref, b2_ref,
              w3t_ref, b3_ref, out_ref):
    c = jnp.concatenate([tl_ref[...], ts_ref[...]], axis=1)
    z1 = _r16(_r16(jnp.dot(c, w1t_ref[...],
                           preferred_element_type=jnp.float32)) + b1_ref[...])
    h1 = jnp.maximum(z1, 0.0)
    z2 = _r16(_r16(jnp.dot(h1, w2t_ref[...],
                           preferred_element_type=jnp.float32)) + b2_ref[...])
    h2 = jnp.maximum(z2, 0.0)
    z3 = _r16(_r16(jnp.dot(h2, w3t_ref[...],
                           preferred_element_type=jnp.float32)) + b3_ref[...])
    raw = _r16(jax.nn.sigmoid(z3))
    s = _r16(jnp.sum(raw, axis=1, keepdims=True))
    out_ref[...] = raw / s


def _topk_stage(llm32, slm32):
    return pl.pallas_call(
        _topk_body,
        grid=(_NG, _NC),
        in_specs=[
            pl.BlockSpec((_RG, _CHUNK), lambda g, c: (g, c)),
            pl.BlockSpec((_RG, _CHUNK), lambda g, c: (g, c)),
        ],
        out_specs=[
            pl.BlockSpec((_RG, _K), lambda g, c: (g, 0)),
            pl.BlockSpec((_RG, _K), lambda g, c: (g, 0)),
        ],
        out_shape=[
            jax.ShapeDtypeStruct((_B, _K), jnp.float32),
            jax.ShapeDtypeStruct((_B, _K), jnp.float32),
        ],
        scratch_shapes=[
            pltpu.VMEM((_RG, _K * _LANES), jnp.float32),
            pltpu.VMEM((_RG, _K * _LANES), jnp.float32),
        ],
        compiler_params=pltpu.CompilerParams(
            dimension_semantics=("parallel", "arbitrary")),
    )(llm32, slm32)


def kernel(llm_logits, slm_logits, W1, b1, W2, b2, W3, b3):
    llm32 = llm_logits.astype(jnp.float32)
    slm32 = slm_logits.astype(jnp.float32)
    tl, ts = _topk_stage(llm32, slm32)

    w1t = W1.T.astype(jnp.float32)
    w2t = W2.T.astype(jnp.float32)
    w3t = W3.T.astype(jnp.float32)
    b1r = b1.reshape(1, -1).astype(jnp.float32)
    b2r = b2.reshape(1, -1).astype(jnp.float32)
    b3r = b3.reshape(1, -1).astype(jnp.float32)

    full = lambda shape: pl.BlockSpec(shape, lambda: (0,) * len(shape))
    out = pl.pallas_call(
        _mlp_body,
        in_specs=[full((_B, _K)), full((_B, _K)),
                  full(w1t.shape), full(b1r.shape),
                  full(w2t.shape), full(b2r.shape),
                  full(w3t.shape), full(b3r.shape)],
        out_specs=full((_B, 2)),
        out_shape=jax.ShapeDtypeStruct((_B, 2), jnp.float32),
    )(tl, ts, w1t, b1r, w2t, b2r, w3t, b3r)
    return out.astype(jnp.float16)


# speedup vs baseline: 1.0392x; 1.0015x over previous
import jax
import jax.numpy as jnp
from jax.experimental import pallas as pl
from jax.experimental.pallas import tpu as pltpu

_K = 10
_B = 128
_V = 32768
_RG = 16
_LANES = 128
_CHUNK = 32768
_NC = _V // _CHUNK
_NSL = _CHUNK // _LANES
_NG = _B // _RG
_NEG = -3.0e38


def _insert_chunk(llm_ref, slm_ref, til_ref, tis_ref, first):
    if first:
        init_l = tuple(jnp.full((_RG, _LANES), _NEG, dtype=jnp.float32)
                       for _ in range(_K))
        init_s = init_l
    else:
        init_l = tuple(til_ref[:, k * _LANES:(k + 1) * _LANES]
                       for k in range(_K))
        init_s = tuple(tis_ref[:, k * _LANES:(k + 1) * _LANES]
                       for k in range(_K))

    def body(i, carry):
        tl, ts = list(carry[0]), list(carry[1])
        vl = llm_ref[:, pl.ds(i * _LANES, _LANES)]
        vs = slm_ref[:, pl.ds(i * _LANES, _LANES)]
        for k in range(_K):
            hl = jnp.maximum(tl[k], vl)
            vl = jnp.minimum(tl[k], vl)
            tl[k] = hl
            hs = jnp.maximum(ts[k], vs)
            vs = jnp.minimum(ts[k], vs)
            ts[k] = hs
        return tuple(tl), tuple(ts)

    tl, ts = jax.lax.fori_loop(0, _NSL, body, (init_l, init_s), unroll=128)
    for k in range(_K):
        til_ref[:, k * _LANES:(k + 1) * _LANES] = tl[k]
        tis_ref[:, k * _LANES:(k + 1) * _LANES] = ts[k]


def _merge_topk(tiles_ref):
    cand = tiles_ref[...]
    idx = jax.lax.broadcasted_iota(jnp.int32, cand.shape, 1)
    outs = []
    for _ in range(_K):
        m = jnp.max(cand, axis=1, keepdims=True)
        eq = cand == m
        pos = jnp.min(jnp.where(eq, idx, _K * _LANES), axis=1, keepdims=True)
        cand = jnp.where(idx == pos, _NEG, cand)
        outs.append(m)
    return jnp.concatenate(outs, axis=1)


def _topk_body(llm_ref, slm_ref, otl_ref, ots_ref, til_ref, tis_ref):
    c = pl.program_id(1)

    @pl.when(c == 0)
    def _():
        _insert_chunk(llm_ref, slm_ref, til_ref, tis_ref, True)

    @pl.when(c != 0)
    def _():
        _insert_chunk(llm_ref, slm_ref, til_ref, tis_ref, False)

    @pl.when(c == _NC - 1)
    def _():
        otl_ref[...] = _merge_topk(til_ref)
        ots_ref[...] = _merge_topk(tis_ref)


def _mlp_body(tl_ref, ts_ref, w1t_ref, b1_ref, w2t_ref, b2_ref,
              w3t_ref, b3_ref, out_ref):
    c = jnp.concatenate([tl_ref[...], ts_ref[...]], axis=1)
    z1 = jnp.dot(c, w1t_ref[...], preferred_element_type=jnp.float32) + b1_ref[...]
    h1 = jnp.maximum(z1, 0.0)
    z2 = jnp.dot(h1, w2t_ref[...], preferred_element_type=jnp.float32) + b2_ref[...]
    h2 = jnp.maximum(z2, 0.0)
    z3 = jnp.dot(h2, w3t_ref[...], preferred_element_type=jnp.float32) + b3_ref[...]
    raw = jax.nn.sigmoid(z3)
    out_ref[...] = raw / jnp.sum(raw, axis=1, keepdims=True)


def _topk_stage(llm32, slm32):
    return pl.pallas_call(
        _topk_body,
        grid=(_NG, _NC),
        in_specs=[
            pl.BlockSpec((_RG, _CHUNK), lambda g, c: (g, c)),
            pl.BlockSpec((_RG, _CHUNK), lambda g, c: (g, c)),
        ],
        out_specs=[
            pl.BlockSpec((_RG, _K), lambda g, c: (g, 0)),
            pl.BlockSpec((_RG, _K), lambda g, c: (g, 0)),
        ],
        out_shape=[
            jax.ShapeDtypeStruct((_B, _K), jnp.float32),
            jax.ShapeDtypeStruct((_B, _K), jnp.float32),
        ],
        scratch_shapes=[
            pltpu.VMEM((_RG, _K * _LANES), jnp.float32),
            pltpu.VMEM((_RG, _K * _LANES), jnp.float32),
        ],
        compiler_params=pltpu.CompilerParams(
            dimension_semantics=("parallel", "arbitrary")),
    )(llm32, slm32)


def kernel(llm_logits, slm_logits, W1, b1, W2, b2, W3, b3):
    llm32 = llm_logits.astype(jnp.float32)
    slm32 = slm_logits.astype(jnp.float32)
    tl, ts = _topk_stage(llm32, slm32)

    w1t = W1.T.astype(jnp.float32)
    w2t = W2.T.astype(jnp.float32)
    w3t = W3.T.astype(jnp.float32)
    b1r = b1.reshape(1, -1).astype(jnp.float32)
    b2r = b2.reshape(1, -1).astype(jnp.float32)
    b3r = b3.reshape(1, -1).astype(jnp.float32)

    full = lambda shape: pl.BlockSpec(shape, lambda: (0,) * len(shape))
    out = pl.pallas_call(
        _mlp_body,
        in_specs=[full((_B, _K)), full((_B, _K)),
                  full(w1t.shape), full(b1r.shape),
                  full(w2t.shape), full(b2r.shape),
                  full(w3t.shape), full(b3r.shape)],
        out_specs=full((_B, 2)),
        out_shape=jax.ShapeDtypeStruct((_B, 2), jnp.float32),
    )(tl, ts, w1t, b1r, w2t, b2r, w3t, b3r)
    return out.astype(jnp.float16)
